# baseline (speedup 1.0000x reference)
import jax
import jax.numpy as jnp
from jax import lax
from jax.experimental import pallas as pl
from jax.experimental.pallas import tpu as pltpu

_Q = 32
_D = 512
_BM = 5000
_K = 5


def _knn_kernel(obs_ref, W_ref, b_ref, mem_ref, out_ref, emb_s, run_s):
    i = pl.program_id(0)
    nb = pl.num_programs(0)

    @pl.when(i == 0)
    def _init():
        emb = lax.dot_general(
            obs_ref[...], W_ref[...], (((1,), (0,)), ((), ())),
            preferred_element_type=jnp.float32)
        emb_s[...] = emb + b_ref[...]
        run_s[...] = jnp.full((_Q, 128), jnp.inf, jnp.float32)

    mem = mem_ref[...]
    emb = emb_s[...].astype(jnp.bfloat16)
    s = lax.dot_general(emb, mem, (((1,), (1,)), ((), ())),
                        preferred_element_type=jnp.float32)
    msq = mem * mem
    ones = jnp.ones((8, _D), jnp.bfloat16)
    m2 = lax.dot_general(ones, msq, (((1,), (1,)), ((), ())),
                         preferred_element_type=jnp.float32)
    t = m2[0:1, :] - 2.0 * s

    v = jnp.concatenate([run_s[...], t], axis=1)
    iota = lax.broadcasted_iota(jnp.int32, v.shape, 1)
    liota = lax.broadcasted_iota(jnp.int32, (_Q, 128), 1)
    newrun = jnp.full((_Q, 128), jnp.inf, jnp.float32)
    for k in range(_K):
        m = jnp.min(v, axis=1, keepdims=True)
        ismin = v == m
        fidx = jnp.min(jnp.where(ismin, iota, jnp.iinfo(jnp.int32).max),
                       axis=1, keepdims=True)
        v = jnp.where(iota == fidx, jnp.inf, v)
        newrun = jnp.where(liota == k, m, newrun)
    run_s[...] = newrun

    @pl.when(i == nb - 1)
    def _fin():
        e = emb_s[...]
        q2 = jnp.sum(e * e, axis=1, keepdims=True)
        d2 = jnp.maximum(run_s[...] + q2, 0.0) + 1e-12
        dist = jnp.sqrt(d2)
        out_ref[0, 0] = jnp.sum(jnp.where(liota < _K, dist, 0.0)) / (_Q * _K)


def kernel(obs, memory, W, b):
    nb = memory.shape[0] // _BM
    b2 = b.reshape(1, _D)
    memory = memory.astype(jnp.bfloat16)
    out = pl.pallas_call(
        _knn_kernel,
        grid=(nb,),
        in_specs=[
            pl.BlockSpec(obs.shape, lambda i: (0, 0)),
            pl.BlockSpec(W.shape, lambda i: (0, 0)),
            pl.BlockSpec((1, _D), lambda i: (0, 0)),
            pl.BlockSpec((_BM, _D), lambda i: (i, 0)),
        ],
        out_specs=pl.BlockSpec((1, 1), lambda i: (0, 0),
                               memory_space=pltpu.SMEM),
        out_shape=jax.ShapeDtypeStruct((1, 1), jnp.float32),
        scratch_shapes=[
            pltpu.VMEM((_Q, _D), jnp.float32),
            pltpu.VMEM((_Q, 128), jnp.float32),
        ],
    )(obs, W, b2, memory)
    return out[0, 0]


# speedup vs baseline: 2.3375x; 2.3375x over previous
import jax
import jax.numpy as jnp
from jax import lax
from jax.experimental import pallas as pl
from jax.experimental.pallas import tpu as pltpu

_Q = 32
_D = 512
_BM = 4096
_M = 100000
_K = 5
_CH = _BM // 128


def _knn_kernel(obs_ref, W_ref, b_ref, mem_ref, out_ref, a_s, emb_s, run_s):
    i = pl.program_id(0)
    nb = pl.num_programs(0)

    @pl.when(i == 0)
    def _init():
        emb = lax.dot_general(
            obs_ref[...], W_ref[...], (((1,), (0,)), ((), ())),
            preferred_element_type=jnp.float32) + b_ref[...]
        emb_s[...] = emb
        a_s[:, :_D] = (-2.0 * emb).astype(jnp.bfloat16)
        a_s[:, _D:] = jnp.ones((_Q, _D), jnp.bfloat16)
        run_s[...] = jnp.full((_Q, _K * 128), jnp.inf, jnp.float32)

    mem = mem_ref[...]
    memb = mem.astype(jnp.bfloat16)
    msq = memb * memb
    bmat = jnp.concatenate([memb, msq], axis=1)
    t = lax.dot_general(a_s[...], bmat, (((1,), (1,)), ((), ())),
                        preferred_element_type=jnp.float32)

    valid = jnp.minimum(_M - i * _BM, _BM)
    iota = lax.broadcasted_iota(jnp.int32, (_Q, _BM), 1)
    t = jnp.where(iota < valid, t, jnp.inf)

    r = [run_s[:, k * 128:(k + 1) * 128] for k in range(_K)]
    for c in range(_CH):
        x = t[:, c * 128:(c + 1) * 128]
        for k in range(_K):
            lo = jnp.minimum(r[k], x)
            x = jnp.maximum(r[k], x)
            r[k] = lo
    for k in range(_K):
        run_s[:, k * 128:(k + 1) * 128] = r[k]

    @pl.when(i == nb - 1)
    def _fin():
        e = emb_s[...]
        q2 = jnp.sum(e * e, axis=1, keepdims=True)
        cand = run_s[...]
        acc = jnp.zeros((_Q, 1), jnp.float32)
        for _ in range(_K):
            m = jnp.min(cand, axis=1, keepdims=True)
            cand = jnp.where(cand == m, jnp.inf, cand)
            acc = acc + jnp.sqrt(jnp.maximum(m + q2, 0.0) + 1e-12)
        out_ref[0, 0] = jnp.sum(acc) / (_Q * _K)


def kernel(obs, memory, W, b):
    nb = pl.cdiv(_M, _BM)
    b2 = b.reshape(1, _D)
    out = pl.pallas_call(
        _knn_kernel,
        grid=(nb,),
        in_specs=[
            pl.BlockSpec(obs.shape, lambda i: (0, 0)),
            pl.BlockSpec(W.shape, lambda i: (0, 0)),
            pl.BlockSpec((1, _D), lambda i: (0, 0)),
            pl.BlockSpec((_BM, _D), lambda i: (i, 0)),
        ],
        out_specs=pl.BlockSpec((1, 1), lambda i: (0, 0),
                               memory_space=pltpu.SMEM),
        out_shape=jax.ShapeDtypeStruct((1, 1), jnp.float32),
        scratch_shapes=[
            pltpu.VMEM((_Q, 2 * _D), jnp.bfloat16),
            pltpu.VMEM((_Q, _D), jnp.float32),
            pltpu.VMEM((_Q, _K * 128), jnp.float32),
        ],
    )(obs, W, b2, memory)
    return out[0, 0]


# speedup vs baseline: 2.3381x; 1.0003x over previous
import jax
import jax.numpy as jnp
from jax import lax
from jax.experimental import pallas as pl
from jax.experimental.pallas import tpu as pltpu

_Q = 32
_D = 512
_BM = 4096
_M = 100000
_K = 5
_SB = 1024


def _knn_kernel(obs_ref, W_ref, b_ref, mem_ref, out_ref, a_s, emb_s, run_s):
    i = pl.program_id(0)
    nb = pl.num_programs(0)

    @pl.when(i == 0)
    def _init():
        emb = lax.dot_general(
            obs_ref[...], W_ref[...], (((1,), (0,)), ((), ())),
            preferred_element_type=jnp.float32) + b_ref[...]
        emb_s[...] = emb
        a_s[:, :_D] = (-2.0 * emb).astype(jnp.bfloat16)
        a_s[:, _D:] = jnp.ones((_Q, _D), jnp.bfloat16)
        run_s[...] = jnp.full((_Q, 2 * _K * 128), jnp.inf, jnp.float32)

    r = [run_s[:, k * 128:(k + 1) * 128] for k in range(2 * _K)]
    valid = _M - i * _BM
    a = a_s[...]
    iota = lax.broadcasted_iota(jnp.int32, (_Q, _SB), 1)
    for g in range(_BM // _SB):
        mem_g = mem_ref[pl.ds(g * _SB, _SB), :]
        memb = mem_g.astype(jnp.bfloat16)
        msq = memb * memb
        bmat = jnp.concatenate([memb, msq], axis=1)
        t = lax.dot_general(a, bmat, (((1,), (1,)), ((), ())),
                            preferred_element_type=jnp.float32)
        t = jnp.where(iota < valid - g * _SB, t, jnp.inf)
        for c in range(_SB // 128):
            x = t[:, c * 128:(c + 1) * 128]
            o = (c % 2) * _K
            for k in range(_K):
                lo = jnp.minimum(r[o + k], x)
                x = jnp.maximum(r[o + k], x)
                r[o + k] = lo
    for k in range(2 * _K):
        run_s[:, k * 128:(k + 1) * 128] = r[k]

    @pl.when(i == nb - 1)
    def _fin():
        e = emb_s[...]
        q2 = jnp.sum(e * e, axis=1, keepdims=True)
        cand = run_s[...]
        acc = jnp.zeros((_Q, 1), jnp.float32)
        for _ in range(_K):
            m = jnp.min(cand, axis=1, keepdims=True)
            cand = jnp.where(cand == m, jnp.inf, cand)
            acc = acc + jnp.sqrt(jnp.maximum(m + q2, 0.0) + 1e-12)
        out_ref[0, 0] = jnp.sum(acc) / (_Q * _K)


def kernel(obs, memory, W, b):
    nb = pl.cdiv(_M, _BM)
    b2 = b.reshape(1, _D)
    out = pl.pallas_call(
        _knn_kernel,
        grid=(nb,),
        in_specs=[
            pl.BlockSpec(obs.shape, lambda i: (0, 0)),
            pl.BlockSpec(W.shape, lambda i: (0, 0)),
            pl.BlockSpec((1, _D), lambda i: (0, 0)),
            pl.BlockSpec((_BM, _D), lambda i: (i, 0)),
        ],
        out_specs=pl.BlockSpec((1, 1), lambda i: (0, 0),
                               memory_space=pltpu.SMEM),
        out_shape=jax.ShapeDtypeStruct((1, 1), jnp.float32),
        scratch_shapes=[
            pltpu.VMEM((_Q, 2 * _D), jnp.bfloat16),
            pltpu.VMEM((_Q, _D), jnp.float32),
            pltpu.VMEM((_Q, 2 * _K * 128), jnp.float32),
        ],
    )(obs, W, b2, memory)
    return out[0, 0]


# speedup vs baseline: 2.3965x; 1.0250x over previous
import jax
import jax.numpy as jnp
from jax import lax
from jax.experimental import pallas as pl
from jax.experimental.pallas import tpu as pltpu

_Q = 32
_D = 512
_BM = 8192
_M = 100000
_K = 5
_SB = 1024


def _knn_kernel(obs_ref, W_ref, b_ref, mem_ref, out_ref, a_s, emb_s, run_s):
    i = pl.program_id(0)
    nb = pl.num_programs(0)

    @pl.when(i == 0)
    def _init():
        emb = lax.dot_general(
            obs_ref[...], W_ref[...], (((1,), (0,)), ((), ())),
            preferred_element_type=jnp.float32) + b_ref[...]
        emb_s[...] = emb
        a_s[:, :_D] = (-2.0 * emb).astype(jnp.bfloat16)
        a_s[:, _D:] = jnp.ones((_Q, _D), jnp.bfloat16)
        run_s[...] = jnp.full((_Q, 2 * _K * 128), jnp.inf, jnp.float32)

    r = [run_s[:, k * 128:(k + 1) * 128] for k in range(2 * _K)]
    valid = _M - i * _BM
    a = a_s[...]
    iota = lax.broadcasted_iota(jnp.int32, (_Q, _SB), 1)
    for g in range(_BM // _SB):
        mem_g = mem_ref[pl.ds(g * _SB, _SB), :]
        memb = mem_g.astype(jnp.bfloat16)
        msq = memb * memb
        bmat = jnp.concatenate([memb, msq], axis=1)
        t = lax.dot_general(a, bmat, (((1,), (1,)), ((), ())),
                            preferred_element_type=jnp.float32)
        t = jnp.where(iota < valid - g * _SB, t, jnp.inf)
        for c in range(_SB // 128):
            x = t[:, c * 128:(c + 1) * 128]
            o = (c % 2) * _K
            for k in range(_K):
                lo = jnp.minimum(r[o + k], x)
                x = jnp.maximum(r[o + k], x)
                r[o + k] = lo
    for k in range(2 * _K):
        run_s[:, k * 128:(k + 1) * 128] = r[k]

    @pl.when(i == nb - 1)
    def _fin():
        e = emb_s[...]
        q2 = jnp.sum(e * e, axis=1, keepdims=True)
        cand = run_s[...]
        acc = jnp.zeros((_Q, 1), jnp.float32)
        for _ in range(_K):
            m = jnp.min(cand, axis=1, keepdims=True)
            cand = jnp.where(cand == m, jnp.inf, cand)
            acc = acc + jnp.sqrt(jnp.maximum(m + q2, 0.0) + 1e-12)
        out_ref[0, 0] = jnp.sum(acc) / (_Q * _K)


def kernel(obs, memory, W, b):
    nb = pl.cdiv(_M, _BM)
    b2 = b.reshape(1, _D)
    out = pl.pallas_call(
        _knn_kernel,
        grid=(nb,),
        in_specs=[
            pl.BlockSpec(obs.shape, lambda i: (0, 0)),
            pl.BlockSpec(W.shape, lambda i: (0, 0)),
            pl.BlockSpec((1, _D), lambda i: (0, 0)),
            pl.BlockSpec((_BM, _D), lambda i: (i, 0)),
        ],
        out_specs=pl.BlockSpec((1, 1), lambda i: (0, 0),
                               memory_space=pltpu.SMEM),
        out_shape=jax.ShapeDtypeStruct((1, 1), jnp.float32),
        scratch_shapes=[
            pltpu.VMEM((_Q, 2 * _D), jnp.bfloat16),
            pltpu.VMEM((_Q, _D), jnp.float32),
            pltpu.VMEM((_Q, 2 * _K * 128), jnp.float32),
        ],
    )(obs, W, b2, memory)
    return out[0, 0]


# speedup vs baseline: 2.4288x; 1.0135x over previous
import jax
import jax.numpy as jnp
from jax import lax
from jax.experimental import pallas as pl
from jax.experimental.pallas import tpu as pltpu

_Q = 32
_D = 512
_M = 100000
_MH = _M // 2
_BM = 4096
_K = 5
_SB = 1024
_NSETS = 4


def _scan(mem_ref, a, r, rbase, valid, iota):
    for g in range(_BM // _SB):
        mem_g = mem_ref[0, pl.ds(g * _SB, _SB), :]
        memb = mem_g.astype(jnp.bfloat16)
        msq = memb * memb
        bmat = jnp.concatenate([memb, msq], axis=1)
        t = lax.dot_general(a, bmat, (((1,), (1,)), ((), ())),
                            preferred_element_type=jnp.float32)
        t = jnp.where(iota < valid - g * _SB, t, jnp.inf)
        for c in range(_SB // 128):
            x = t[:, c * 128:(c + 1) * 128]
            o = rbase + (c % 2) * _K
            for k in range(_K):
                lo = jnp.minimum(r[o + k], x)
                x = jnp.maximum(r[o + k], x)
                r[o + k] = lo


def _knn_kernel(obs_ref, W_ref, b_ref, memA_ref, memB_ref, out_ref,
                a_s, emb_s, run_s):
    i = pl.program_id(0)
    nb = pl.num_programs(0)

    @pl.when(i == 0)
    def _init():
        emb = lax.dot_general(
            obs_ref[...], W_ref[...], (((1,), (0,)), ((), ())),
            preferred_element_type=jnp.float32) + b_ref[...]
        emb_s[...] = emb
        a_s[:, :_D] = (-2.0 * emb).astype(jnp.bfloat16)
        a_s[:, _D:] = jnp.ones((_Q, _D), jnp.bfloat16)
        run_s[...] = jnp.full((_Q, _NSETS * _K * 128), jnp.inf, jnp.float32)

    r = [run_s[:, k * 128:(k + 1) * 128] for k in range(_NSETS * _K)]
    valid = _MH - i * _BM
    a = a_s[...]
    iota = lax.broadcasted_iota(jnp.int32, (_Q, _SB), 1)
    _scan(memA_ref, a, r, 0, valid, iota)
    _scan(memB_ref, a, r, 2 * _K, valid, iota)
    for k in range(_NSETS * _K):
        run_s[:, k * 128:(k + 1) * 128] = r[k]

    @pl.when(i == nb - 1)
    def _fin():
        e = emb_s[...]
        q2 = jnp.sum(e * e, axis=1, keepdims=True)
        cand = run_s[...]
        acc = jnp.zeros((_Q, 1), jnp.float32)
        for _ in range(_K):
            m = jnp.min(cand, axis=1, keepdims=True)
            cand = jnp.where(cand == m, jnp.inf, cand)
            acc = acc + jnp.sqrt(jnp.maximum(m + q2, 0.0) + 1e-12)
        out_ref[0, 0] = jnp.sum(acc) / (_Q * _K)


def kernel(obs, memory, W, b):
    nb = pl.cdiv(_MH, _BM)
    b2 = b.reshape(1, _D)
    mem3 = memory.reshape(2, _MH, _D)
    out = pl.pallas_call(
        _knn_kernel,
        grid=(nb,),
        in_specs=[
            pl.BlockSpec(obs.shape, lambda i: (0, 0)),
            pl.BlockSpec(W.shape, lambda i: (0, 0)),
            pl.BlockSpec((1, _D), lambda i: (0, 0)),
            pl.BlockSpec((1, _BM, _D), lambda i: (0, i, 0)),
            pl.BlockSpec((1, _BM, _D), lambda i: (1, i, 0)),
        ],
        out_specs=pl.BlockSpec((1, 1), lambda i: (0, 0),
                               memory_space=pltpu.SMEM),
        out_shape=jax.ShapeDtypeStruct((1, 1), jnp.float32),
        scratch_shapes=[
            pltpu.VMEM((_Q, 2 * _D), jnp.bfloat16),
            pltpu.VMEM((_Q, _D), jnp.float32),
            pltpu.VMEM((_Q, _NSETS * _K * 128), jnp.float32),
        ],
    )(obs, W, b2, mem3, mem3)
    return out[0, 0]
